# baseline (speedup 1.0000x reference)
import numpy as np
import jax
import jax.numpy as jnp
from jax.experimental import pallas as pl
from jax.experimental.pallas import tpu as pltpu

IMG_TILE = 1024
CHUNK = 16

_PERM = np.array([(j % 32) * 3 + (j // 32) for j in range(96)], dtype=np.int32)

_E1 = np.zeros((112, 256), np.float32)
_E1B = np.zeros((112, 256), np.float32)
for _i in range(8):
    for _r in range(14):
        _E1[_i * 14 + _r, _i * 32 + 2 * _r] = 1.0
        _E1B[_i * 14 + _r, _i * 32 + 2 * _r + 1] = 1.0
_E3B = np.zeros((CHUNK, CHUNK * 14), np.float32)
for _i in range(CHUNK):
    _E3B[_i, _i * 14] = 1.0


def _evshift(vb, k):
    vi = pltpu.bitcast(vb, jnp.int32)
    vi = pltpu.roll(vi, (vi.shape[0] - k) % vi.shape[0], axis=0)
    return pltpu.bitcast(vi, jnp.bfloat16)


def _dot3(pieces, w_ref):
    f32 = jnp.float32
    acc = jnp.dot(jnp.concatenate(pieces[0:2], axis=1), w_ref[0:256, :],
                  preferred_element_type=f32)
    acc = acc + jnp.dot(jnp.concatenate(pieces[2:4], axis=1), w_ref[256:512, :],
                        preferred_element_type=f32)
    return acc + jnp.dot(pieces[4], w_ref[512:640, :], preferred_element_type=f32)


def _fused_kernel(x_ref, w1c_ref, b1_ref, e1_ref, w2c_ref, b2_ref,
                  wf1c_ref, bf1_ref, e3b_ref, wf2_ref, bf2_ref,
                  wf3_ref, bf3_ref, out_ref):
    f32, bf16 = jnp.float32, jnp.bfloat16
    e1 = e1_ref[0]
    e1b = e1_ref[1]
    e3b = e3b_ref[...]
    nc = IMG_TILE // CHUNK
    r1 = CHUNK * 32
    r2 = CHUNK * 14

    def pad128(v):
        return jnp.pad(v, ((0, 0), (0, 128 - v.shape[1])))

    xs = [x_ref[c * CHUNK:(c + 1) * CHUNK].reshape(r1, 96) for c in range(nc)]
    xbs = []
    for x in xs:
        x0p = pad128(x.astype(bf16))
        x1p = pad128(pltpu.roll(x, r1 - 1, axis=0).astype(bf16))
        xbs.append([x0p, x1p, _evshift(x0p, 1), _evshift(x1p, 1),
                    _evshift(x0p, 2)])
    acc1s = [_dot3(xb, w1c_ref) for xb in xbs]

    m1s = []
    for acc1 in acc1s:
        yw = jnp.maximum(jnp.maximum(acc1[:, :128], acc1[:, 128:])
                         + b1_ref[...], 0.0)
        m1s.append(jnp.maximum(yw, pltpu.roll(yw, r1 - 1, axis=0)).astype(bf16))
    p1s = []
    for m1 in m1s:
        mcat = jnp.concatenate([m1[0:256, :], m1[256:512, :]], axis=1)
        pp = jnp.dot(e1, mcat, preferred_element_type=f32)
        p1s.append(jnp.concatenate([pp[:, :128], pp[:, 128:]], axis=0))

    pbs = []
    for p1 in p1s:
        p0b = p1.astype(bf16)
        p1b = pltpu.roll(p1, r2 - 1, axis=0).astype(bf16)
        pbs.append([p0b, p1b, _evshift(p0b, 1), _evshift(p1b, 1),
                    _evshift(p0b, 2)])
    acc2s = [_dot3(pb, w2c_ref) for pb in pbs]

    fbs = []
    for acc2 in acc2s:
        zw = jnp.maximum(jnp.maximum(acc2[:, :128], acc2[:, 128:])
                         + b2_ref[...], 0.0)
        m2 = jnp.maximum(zw, pltpu.roll(zw, r2 - 1, axis=0))
        m0b = m2.astype(bf16)
        fbs.append([m0b] + [_evshift(m0b, h) for h in range(1, 5)])
    accfs = [_dot3(fb, wf1c_ref) for fb in fbs]

    z1cs = []
    for accf in accfs:
        z1 = jnp.maximum(accf + bf1_ref[...], 0.0)
        z1cs.append(jnp.dot(e3b, z1.astype(bf16), preferred_element_type=f32))
    z1c = jnp.concatenate(z1cs, axis=0)
    z2 = jnp.maximum(jnp.dot(z1c.astype(bf16), wf2_ref[...],
                             preferred_element_type=f32) + bf2_ref[...], 0.0)
    z3 = jnp.dot(z2.astype(bf16), wf3_ref[...],
                 preferred_element_type=f32) + bf3_ref[...]
    out_ref[...] = z3


def kernel(w1, b1, e1, w2, b2, wf1, bf1, e3, wf2, bf2, wf3, bf3, x_nchw):
    T = IMG_TILE
    N = x_nchw.shape[0]
    n_pad = (-N) % T
    Np = N + n_pad

    x3d = jnp.transpose(x_nchw, (0, 2, 1, 3)).reshape(N, 32, 96)
    if n_pad:
        x3d = jnp.pad(x3d, ((0, n_pad), (0, 0), (0, 0)))

    w1c = jnp.zeros((5, 128, 256), jnp.bfloat16).at[:, :96, :].set(w1[:, _PERM, :])
    w1c = w1c.reshape(640, 256)
    w2c = w2.reshape(640, 256)
    wf1c = wf1.reshape(640, 128)
    e1c = jnp.asarray(np.stack([_E1, _E1B]), jnp.bfloat16)
    e3b = jnp.asarray(_E3B, jnp.bfloat16)

    def full(*shape):
        return pl.BlockSpec(shape, lambda n, _s=len(shape): (0,) * _s)

    out = pl.pallas_call(
        _fused_kernel,
        out_shape=jax.ShapeDtypeStruct((Np, 128), jnp.float32),
        grid=(Np // T,),
        in_specs=[
            pl.BlockSpec((T, 32, 96), lambda n: (n, 0, 0)),
            full(640, 256), full(1, 128), full(2, 112, 256),
            full(640, 256), full(1, 128),
            full(640, 128), full(1, 128), full(CHUNK, CHUNK * 14),
            full(128, 128), full(1, 128),
            full(128, 128), full(1, 128),
        ],
        out_specs=pl.BlockSpec((T, 128), lambda n: (n, 0)),
        compiler_params=pltpu.CompilerParams(
            dimension_semantics=("parallel",)),
    )(x3d, w1c, b1, e1c, w2c, b2, wf1c, bf1, e3b, wf2, bf2, wf3, bf3)
    return out[:N, :10]


# speedup vs baseline: 1.0088x; 1.0088x over previous
import numpy as np
import jax
import jax.numpy as jnp
from jax.experimental import pallas as pl
from jax.experimental.pallas import tpu as pltpu

IMG_TILE = 512
CHUNK = 16

_PERM = np.array([(j % 32) * 3 + (j // 32) for j in range(96)], dtype=np.int32)

_E1 = np.zeros((112, 256), np.float32)
_E1B = np.zeros((112, 256), np.float32)
for _i in range(8):
    for _r in range(14):
        _E1[_i * 14 + _r, _i * 32 + 2 * _r] = 1.0
        _E1B[_i * 14 + _r, _i * 32 + 2 * _r + 1] = 1.0
_E3B = np.zeros((CHUNK, CHUNK * 14), np.float32)
for _i in range(CHUNK):
    _E3B[_i, _i * 14] = 1.0


def _evshift(vb, k):
    vi = pltpu.bitcast(vb, jnp.int32)
    vi = pltpu.roll(vi, (vi.shape[0] - k) % vi.shape[0], axis=0)
    return pltpu.bitcast(vi, jnp.bfloat16)


def _dot3(pieces, w_ref):
    f32 = jnp.float32
    acc = jnp.dot(jnp.concatenate(pieces[0:2], axis=1), w_ref[0:256, :],
                  preferred_element_type=f32)
    acc = acc + jnp.dot(jnp.concatenate(pieces[2:4], axis=1), w_ref[256:512, :],
                        preferred_element_type=f32)
    return acc + jnp.dot(pieces[4], w_ref[512:640, :], preferred_element_type=f32)


def _fused_kernel(x_ref, w1c_ref, b1_ref, e1_ref, w2c_ref, b2_ref,
                  wf1c_ref, bf1_ref, e3b_ref, wf2_ref, bf2_ref,
                  wf3_ref, bf3_ref, out_ref):
    f32, bf16 = jnp.float32, jnp.bfloat16
    e1 = e1_ref[0]
    e1b = e1_ref[1]
    e3b = e3b_ref[...]
    nc = IMG_TILE // CHUNK
    r1 = CHUNK * 32
    r2 = CHUNK * 14

    def pad128(v):
        return jnp.pad(v, ((0, 0), (0, 128 - v.shape[1])))

    xs = [x_ref[c * CHUNK:(c + 1) * CHUNK].reshape(r1, 96) for c in range(nc)]
    xbs = []
    for x in xs:
        x0p = pad128(x.astype(bf16))
        x1p = pad128(pltpu.roll(x, r1 - 1, axis=0).astype(bf16))
        xbs.append([x0p, x1p, _evshift(x0p, 1), _evshift(x1p, 1),
                    _evshift(x0p, 2)])
    acc1s = [_dot3(xb, w1c_ref) for xb in xbs]

    m1s = []
    for acc1 in acc1s:
        yw = jnp.maximum(jnp.maximum(acc1[:, :128], acc1[:, 128:])
                         + b1_ref[...], 0.0)
        m1s.append(jnp.maximum(yw, pltpu.roll(yw, r1 - 1, axis=0)).astype(bf16))
    p1s = []
    for m1 in m1s:
        mcat = jnp.concatenate([m1[0:256, :], m1[256:512, :]], axis=1)
        pp = jnp.dot(e1, mcat, preferred_element_type=f32)
        p1s.append(jnp.concatenate([pp[:, :128], pp[:, 128:]], axis=0))

    pbs = []
    for p1 in p1s:
        p0b = p1.astype(bf16)
        p1b = pltpu.roll(p1, r2 - 1, axis=0).astype(bf16)
        pbs.append([p0b, p1b, _evshift(p0b, 1), _evshift(p1b, 1),
                    _evshift(p0b, 2)])
    acc2s = [_dot3(pb, w2c_ref) for pb in pbs]

    fbs = []
    for acc2 in acc2s:
        zw = jnp.maximum(jnp.maximum(acc2[:, :128], acc2[:, 128:])
                         + b2_ref[...], 0.0)
        m2 = jnp.maximum(zw, pltpu.roll(zw, r2 - 1, axis=0))
        m0b = m2.astype(bf16)
        fbs.append([m0b] + [_evshift(m0b, h) for h in range(1, 5)])
    accfs = [_dot3(fb, wf1c_ref) for fb in fbs]

    z1cs = []
    for accf in accfs:
        z1 = jnp.maximum(accf + bf1_ref[...], 0.0)
        z1cs.append(jnp.dot(e3b, z1.astype(bf16), preferred_element_type=f32))
    z1c = jnp.concatenate(z1cs, axis=0)
    z2 = jnp.maximum(jnp.dot(z1c.astype(bf16), wf2_ref[...],
                             preferred_element_type=f32) + bf2_ref[...], 0.0)
    z3 = jnp.dot(z2.astype(bf16), wf3_ref[...],
                 preferred_element_type=f32) + bf3_ref[...]
    out_ref[...] = z3


def kernel(w1, b1, e1, w2, b2, wf1, bf1, e3, wf2, bf2, wf3, bf3, x_nchw):
    T = IMG_TILE
    N = x_nchw.shape[0]
    n_pad = (-N) % T
    Np = N + n_pad

    x3d = jnp.transpose(x_nchw, (0, 2, 1, 3)).reshape(N, 32, 96)
    if n_pad:
        x3d = jnp.pad(x3d, ((0, n_pad), (0, 0), (0, 0)))

    w1c = jnp.zeros((5, 128, 256), jnp.bfloat16).at[:, :96, :].set(w1[:, _PERM, :])
    w1c = w1c.reshape(640, 256)
    w2c = w2.reshape(640, 256)
    wf1c = wf1.reshape(640, 128)
    e1c = jnp.asarray(np.stack([_E1, _E1B]), jnp.bfloat16)
    e3b = jnp.asarray(_E3B, jnp.bfloat16)

    def full(*shape):
        return pl.BlockSpec(shape, lambda n, _s=len(shape): (0,) * _s)

    out = pl.pallas_call(
        _fused_kernel,
        out_shape=jax.ShapeDtypeStruct((Np, 128), jnp.float32),
        grid=(Np // T,),
        in_specs=[
            pl.BlockSpec((T, 32, 96), lambda n: (n, 0, 0)),
            full(640, 256), full(1, 128), full(2, 112, 256),
            full(640, 256), full(1, 128),
            full(640, 128), full(1, 128), full(CHUNK, CHUNK * 14),
            full(128, 128), full(1, 128),
            full(128, 128), full(1, 128),
        ],
        out_specs=pl.BlockSpec((T, 128), lambda n: (n, 0)),
        compiler_params=pltpu.CompilerParams(
            dimension_semantics=("parallel",)),
    )(x3d, w1c, b1, e1c, w2c, b2, wf1c, bf1, e3b, wf2, bf2, wf3, bf3)
    return out[:N, :10]


# speedup vs baseline: 1.2433x; 1.2325x over previous
import numpy as np
import jax
import jax.numpy as jnp
from jax.experimental import pallas as pl
from jax.experimental.pallas import tpu as pltpu

IMG_TILE = 512
CHUNK = 16

_PERM = np.array([(j % 32) * 3 + (j // 32) for j in range(96)], dtype=np.int32)

_E1 = np.zeros((112, 256), np.float32)
_E1B = np.zeros((112, 256), np.float32)
for _i in range(8):
    for _r in range(14):
        _E1[_i * 14 + _r, _i * 32 + 2 * _r] = 1.0
        _E1B[_i * 14 + _r, _i * 32 + 2 * _r + 1] = 1.0
_E3B = np.zeros((CHUNK, CHUNK * 14), np.float32)
for _i in range(CHUNK):
    _E3B[_i, _i * 14] = 1.0


def _evshift(vb, k):
    vi = pltpu.bitcast(vb, jnp.int32)
    vi = pltpu.roll(vi, (vi.shape[0] - k) % vi.shape[0], axis=0)
    return pltpu.bitcast(vi, jnp.bfloat16)


def _dot3(pieces, w_ref):
    f32 = jnp.float32
    acc = jnp.dot(jnp.concatenate(pieces[0:2], axis=1), w_ref[0:256, :],
                  preferred_element_type=f32)
    acc = acc + jnp.dot(jnp.concatenate(pieces[2:4], axis=1), w_ref[256:512, :],
                        preferred_element_type=f32)
    return acc + jnp.dot(pieces[4], w_ref[512:640, :], preferred_element_type=f32)


def _fused_kernel(x_ref, w1c_ref, b1_ref, e1_ref, w2c_ref, b2_ref,
                  wf1c_ref, bf1_ref, e3b_ref, wf2_ref, bf2_ref,
                  wf3_ref, bf3_ref, out_ref):
    f32, bf16 = jnp.float32, jnp.bfloat16
    e1 = e1_ref[0]
    e1b = e1_ref[1]
    e3b = e3b_ref[...]
    nc = IMG_TILE // CHUNK
    r1 = CHUNK * 32
    r2 = CHUNK * 14

    def pad128(v):
        return jnp.pad(v, ((0, 0), (0, 128 - v.shape[1])))

    xs = [x_ref[c * CHUNK:(c + 1) * CHUNK].reshape(r1, 96) for c in range(nc)]
    xbs = []
    for x in xs:
        x0p = pad128(x)
        x1p = pad128(jnp.concatenate([x[1:], x[:1]], axis=0))
        xbs.append([x0p, x1p, _evshift(x0p, 1), _evshift(x1p, 1),
                    _evshift(x0p, 2)])
    acc1s = [_dot3(xb, w1c_ref) for xb in xbs]

    m1s = []
    for acc1 in acc1s:
        yw = jnp.maximum(jnp.maximum(acc1[:, :128], acc1[:, 128:])
                         + b1_ref[...], 0.0)
        m1s.append(jnp.maximum(yw, pltpu.roll(yw, r1 - 1, axis=0)).astype(bf16))
    p1s = []
    for m1 in m1s:
        mcat = jnp.concatenate([m1[0:256, :], m1[256:512, :]], axis=1)
        pp = jnp.dot(e1, mcat, preferred_element_type=f32)
        p1s.append(jnp.concatenate([pp[:, :128], pp[:, 128:]], axis=0))

    pbs = []
    for p1 in p1s:
        p0b = p1.astype(bf16)
        p1b = pltpu.roll(p1, r2 - 1, axis=0).astype(bf16)
        pbs.append([p0b, p1b, _evshift(p0b, 1), _evshift(p1b, 1),
                    _evshift(p0b, 2)])
    acc2s = [_dot3(pb, w2c_ref) for pb in pbs]

    fbs = []
    for acc2 in acc2s:
        zw = jnp.maximum(jnp.maximum(acc2[:, :128], acc2[:, 128:])
                         + b2_ref[...], 0.0)
        m2 = jnp.maximum(zw, pltpu.roll(zw, r2 - 1, axis=0))
        m0b = m2.astype(bf16)
        fbs.append([m0b] + [_evshift(m0b, h) for h in range(1, 5)])
    accfs = [_dot3(fb, wf1c_ref) for fb in fbs]

    z1cs = []
    for accf in accfs:
        z1 = jnp.maximum(accf + bf1_ref[...], 0.0)
        z1cs.append(jnp.dot(e3b, z1.astype(bf16), preferred_element_type=f32))
    z1c = jnp.concatenate(z1cs, axis=0)
    z2 = jnp.maximum(jnp.dot(z1c.astype(bf16), wf2_ref[...],
                             preferred_element_type=f32) + bf2_ref[...], 0.0)
    z3 = jnp.dot(z2.astype(bf16), wf3_ref[...],
                 preferred_element_type=f32) + bf3_ref[...]
    out_ref[...] = z3


def kernel(w1, b1, e1, w2, b2, wf1, bf1, e3, wf2, bf2, wf3, bf3, x_nchw):
    T = IMG_TILE
    N = x_nchw.shape[0]
    n_pad = (-N) % T
    Np = N + n_pad

    x3d = jnp.transpose(x_nchw, (0, 2, 1, 3)).reshape(N, 32, 96).astype(jnp.bfloat16)
    if n_pad:
        x3d = jnp.pad(x3d, ((0, n_pad), (0, 0), (0, 0)))

    w1c = jnp.zeros((5, 128, 256), jnp.bfloat16).at[:, :96, :].set(w1[:, _PERM, :])
    w1c = w1c.reshape(640, 256)
    w2c = w2.reshape(640, 256)
    wf1c = wf1.reshape(640, 128)
    e1c = jnp.asarray(np.stack([_E1, _E1B]), jnp.bfloat16)
    e3b = jnp.asarray(_E3B, jnp.bfloat16)

    def full(*shape):
        return pl.BlockSpec(shape, lambda n, _s=len(shape): (0,) * _s)

    out = pl.pallas_call(
        _fused_kernel,
        out_shape=jax.ShapeDtypeStruct((Np, 128), jnp.float32),
        grid=(Np // T,),
        in_specs=[
            pl.BlockSpec((T, 32, 96), lambda n: (n, 0, 0)),
            full(640, 256), full(1, 128), full(2, 112, 256),
            full(640, 256), full(1, 128),
            full(640, 128), full(1, 128), full(CHUNK, CHUNK * 14),
            full(128, 128), full(1, 128),
            full(128, 128), full(1, 128),
        ],
        out_specs=pl.BlockSpec((T, 128), lambda n: (n, 0)),
        compiler_params=pltpu.CompilerParams(
            dimension_semantics=("parallel",)),
    )(x3d, w1c, b1, e1c, w2c, b2, wf1c, bf1, e3b, wf2, bf2, wf3, bf3)
    return out[:N, :10]


# speedup vs baseline: 1.2490x; 1.0046x over previous
import numpy as np
import jax
import jax.numpy as jnp
from jax.experimental import pallas as pl
from jax.experimental.pallas import tpu as pltpu

IMG_TILE = 512
CHUNK = 16

_PERM = np.array([(j % 32) * 3 + (j // 32) for j in range(96)] + [96] * 32,
                 dtype=np.int32)

_E1 = np.zeros((112, 256), np.float32)
_E1B = np.zeros((112, 256), np.float32)
for _i in range(8):
    for _r in range(14):
        _E1[_i * 14 + _r, _i * 32 + 2 * _r] = 1.0
        _E1B[_i * 14 + _r, _i * 32 + 2 * _r + 1] = 1.0
_E3B = np.zeros((CHUNK, CHUNK * 14), np.float32)
for _i in range(CHUNK):
    _E3B[_i, _i * 14] = 1.0


def _evshift(vb, k):
    vi = pltpu.bitcast(vb, jnp.int32)
    vi = pltpu.roll(vi, (vi.shape[0] - k) % vi.shape[0], axis=0)
    return pltpu.bitcast(vi, jnp.bfloat16)


def _dot3(pieces, w_ref):
    f32 = jnp.float32
    acc = jnp.dot(jnp.concatenate(pieces[0:2], axis=1), w_ref[0:256, :],
                  preferred_element_type=f32)
    acc = acc + jnp.dot(jnp.concatenate(pieces[2:4], axis=1), w_ref[256:512, :],
                        preferred_element_type=f32)
    return acc + jnp.dot(pieces[4], w_ref[512:640, :], preferred_element_type=f32)


def _fused_kernel(x_ref, w1c_ref, b1_ref, e1_ref, w2c_ref, b2_ref,
                  wf1c_ref, bf1_ref, e3b_ref, wf2_ref, bf2_ref,
                  wf3_ref, bf3_ref, out_ref):
    f32, bf16 = jnp.float32, jnp.bfloat16
    e1 = e1_ref[0]
    e1b = e1_ref[1]
    e3b = e3b_ref[...]
    nc = IMG_TILE // CHUNK
    r1 = CHUNK * 32
    r2 = CHUNK * 14

    def pad128(v):
        return jnp.pad(v, ((0, 0), (0, 128 - v.shape[1])))

    xs = [x_ref[c * CHUNK:(c + 1) * CHUNK].reshape(r1, 96) for c in range(nc)]
    xbs = []
    for x in xs:
        x0p = pad128(x)
        x1p = pad128(jnp.concatenate([x[1:], x[:1]], axis=0))
        xbs.append([x0p, x1p, _evshift(x0p, 1), _evshift(x1p, 1),
                    _evshift(x0p, 2)])
    acc1s = [_dot3(xb, w1c_ref) for xb in xbs]

    m1s = []
    for acc1 in acc1s:
        yw = jnp.maximum(jnp.maximum(acc1[:, :128], acc1[:, 128:])
                         + b1_ref[...], 0.0)
        m1s.append(jnp.maximum(yw, pltpu.roll(yw, r1 - 1, axis=0)).astype(bf16))
    p1s = []
    for m1 in m1s:
        mcat = jnp.concatenate([m1[0:256, :], m1[256:512, :]], axis=1)
        pp = jnp.dot(e1, mcat, preferred_element_type=f32)
        p1s.append(jnp.concatenate([pp[:, :128], pp[:, 128:]], axis=0))

    pbs = []
    for p1 in p1s:
        p0b = p1.astype(bf16)
        p1b = pltpu.roll(p1, r2 - 1, axis=0).astype(bf16)
        pbs.append([p0b, p1b, _evshift(p0b, 1), _evshift(p1b, 1),
                    _evshift(p0b, 2)])
    acc2s = [_dot3(pb, w2c_ref) for pb in pbs]

    fbs = []
    for acc2 in acc2s:
        zw = jnp.maximum(jnp.maximum(acc2[:, :128], acc2[:, 128:])
                         + b2_ref[...], 0.0)
        m2 = jnp.maximum(zw, pltpu.roll(zw, r2 - 1, axis=0))
        m0b = m2.astype(bf16)
        fbs.append([m0b] + [_evshift(m0b, h) for h in range(1, 5)])
    accfs = [_dot3(fb, wf1c_ref) for fb in fbs]

    z1cs = []
    for accf in accfs:
        z1 = jnp.maximum(accf + bf1_ref[...], 0.0)
        z1cs.append(jnp.dot(e3b, z1.astype(bf16), preferred_element_type=f32))
    z1c = jnp.concatenate(z1cs, axis=0)
    z2 = jnp.maximum(jnp.dot(z1c.astype(bf16), wf2_ref[...],
                             preferred_element_type=f32) + bf2_ref[...], 0.0)
    z3 = jnp.dot(z2.astype(bf16), wf3_ref[...],
                 preferred_element_type=f32) + bf3_ref[...]
    out_ref[...] = z3


def kernel(w1, b1, e1, w2, b2, wf1, bf1, e3, wf2, bf2, wf3, bf3, x_nchw):
    T = IMG_TILE
    N = x_nchw.shape[0]
    n_pad = (-N) % T
    Np = N + n_pad

    x3d = jnp.transpose(x_nchw, (0, 2, 1, 3)).reshape(N, 32, 96).astype(jnp.bfloat16)
    if n_pad:
        x3d = jnp.pad(x3d, ((0, n_pad), (0, 0), (0, 0)))

    w1c = jnp.take(w1, _PERM, axis=1, mode='fill', fill_value=0).reshape(640, 256)
    w2c = w2.reshape(640, 256)
    wf1c = wf1.reshape(640, 128)
    e1c = jnp.asarray(np.stack([_E1, _E1B]), jnp.bfloat16)
    e3b = jnp.asarray(_E3B, jnp.bfloat16)

    def full(*shape):
        return pl.BlockSpec(shape, lambda n, _s=len(shape): (0,) * _s)

    out = pl.pallas_call(
        _fused_kernel,
        out_shape=jax.ShapeDtypeStruct((Np, 128), jnp.float32),
        grid=(Np // T,),
        in_specs=[
            pl.BlockSpec((T, 32, 96), lambda n: (n, 0, 0)),
            full(640, 256), full(1, 128), full(2, 112, 256),
            full(640, 256), full(1, 128),
            full(640, 128), full(1, 128), full(CHUNK, CHUNK * 14),
            full(128, 128), full(1, 128),
            full(128, 128), full(1, 128),
        ],
        out_specs=pl.BlockSpec((T, 128), lambda n: (n, 0)),
        compiler_params=pltpu.CompilerParams(
            dimension_semantics=("parallel",)),
    )(x3d, w1c, b1, e1c, w2c, b2, wf1c, bf1, e3b, wf2, bf2, wf3, bf3)
    return out[:N, :10]
